# baseline (speedup 1.0000x reference)
import functools

import jax
import jax.numpy as jnp
from jax import lax
from jax.experimental import pallas as pl
from jax.experimental.pallas import tpu as pltpu
from jax.experimental.pallas import tpu_sc as plsc

_NUM_FIELDS = 26
_BATCH = 4096
_SEQ = 20
_NUM_BINS = 1000

_ROWS = _NUM_FIELDS * _BATCH
_NW = 32
_ROWS_PER_W = _ROWS // _NW
_CHUNK = 32
_CHUNKS = _ROWS_PER_W // _CHUNK
_CHUNK_ELEMS = _CHUNK * _SEQ
_NVEC = _CHUNK_ELEMS // 16


def _hash16(x):
    h = x.astype(jnp.uint32)
    h = h * jnp.uint32(2654435761)
    h = h ^ (h >> 16)
    h = h * jnp.uint32(2246822519)
    h = h ^ (h >> 13)
    return (h % jnp.uint32(_NUM_BINS)).astype(jnp.int32)


@functools.partial(
    pl.kernel,
    mesh=plsc.VectorSubcoreMesh(core_axis_name="c", subcore_axis_name="s"),
    out_type=jax.ShapeDtypeStruct((_NUM_FIELDS, _BATCH, _NUM_BINS), jnp.float32),
    scratch_types=[
        pltpu.VMEM((_CHUNK_ELEMS,), jnp.int32),
        pltpu.VMEM((_CHUNK_ELEMS,), jnp.int32),
        pltpu.VMEM((_CHUNK, _NUM_BINS), jnp.float32),
    ],
    compiler_params=pltpu.CompilerParams(needs_layout_passes=False),
)
def _multi_hot(in_hbm, out_hbm, in_v, col_v, buf_v):
    wid = lax.axis_index("s") * 2 + lax.axis_index("c")
    base_row = wid * _ROWS_PER_W
    lane = lax.iota(jnp.int32, 16)
    ones = jnp.ones((16,), jnp.float32)
    zeros = jnp.zeros((16,), jnp.float32)

    def zero_row(r, carry):
        def zero_col(j, carry2):
            buf_v[r, pl.ds(j * 16, 16)] = zeros
            return carry2

        lax.fori_loop(0, _NUM_BINS // 16, zero_col, 0, unroll=8)
        buf_v[r, pl.ds(_NUM_BINS - 16, 16)] = zeros
        return carry

    lax.fori_loop(0, _CHUNK, zero_row, 0)

    def chunk_body(c, carry):
        row0 = base_row + c * _CHUNK
        pltpu.sync_copy(in_hbm.at[pl.ds(row0 * _SEQ, _CHUNK_ELEMS)], in_v)

        def hash_body(i, e_vec):
            x = in_v[pl.ds(i * 16, 16)]
            col = _hash16(x)
            row = e_vec // _SEQ
            plsc.store_scatter(buf_v, [row, col], ones)
            col_v[pl.ds(i * 16, 16)] = col
            return e_vec + 16

        lax.fori_loop(0, _NVEC, hash_body, lane, unroll=5)

        f = row0 // _BATCH
        r = row0 % _BATCH
        pltpu.sync_copy(buf_v, out_hbm.at[f, pl.ds(r, _CHUNK), :])

        def reset_body(i, e_vec):
            col = col_v[pl.ds(i * 16, 16)]
            row = e_vec // _SEQ
            plsc.store_scatter(buf_v, [row, col], zeros)
            return e_vec + 16

        lax.fori_loop(0, _NVEC, reset_body, lane, unroll=5)
        return carry

    lax.fori_loop(0, _CHUNKS, chunk_body, 0)


def kernel(inputs):
    flat = inputs.reshape(-1)
    return _multi_hot(flat)


# speedup vs baseline: 1.4817x; 1.4817x over previous
import functools

import jax
import jax.numpy as jnp
from jax import lax
from jax.experimental import pallas as pl
from jax.experimental.pallas import tpu as pltpu
from jax.experimental.pallas import tpu_sc as plsc

_NUM_FIELDS = 26
_BATCH = 4096
_SEQ = 20
_NUM_BINS = 1000

_ROWS = _NUM_FIELDS * _BATCH
_NW = 32
_ROWS_PER_W = _ROWS // _NW
_CHUNK = 32
_CHUNKS = _ROWS_PER_W // _CHUNK
_CHUNK_ELEMS = _CHUNK * _SEQ
_NVEC = _CHUNK_ELEMS // 16


def _hash16(x):
    h = x.astype(jnp.uint32)
    h = h * jnp.uint32(2654435761)
    h = h ^ (h >> 16)
    h = h * jnp.uint32(2246822519)
    h = h ^ (h >> 13)
    return (h % jnp.uint32(_NUM_BINS)).astype(jnp.int32)


@functools.partial(
    pl.kernel,
    mesh=plsc.VectorSubcoreMesh(core_axis_name="c", subcore_axis_name="s"),
    out_type=jax.ShapeDtypeStruct((_ROWS, _NUM_BINS), jnp.float32),
    scratch_types=[
        pltpu.VMEM((_CHUNK_ELEMS,), jnp.int32),
        pltpu.VMEM((_CHUNK_ELEMS,), jnp.int32),
        pltpu.VMEM((_CHUNK_ELEMS,), jnp.int32),
        pltpu.VMEM((_CHUNK_ELEMS,), jnp.int32),
        pltpu.VMEM((_CHUNK, _NUM_BINS), jnp.float32),
        pltpu.VMEM((_CHUNK, _NUM_BINS), jnp.float32),
        pltpu.SemaphoreType.DMA,
        pltpu.SemaphoreType.DMA,
        pltpu.SemaphoreType.DMA,
        pltpu.SemaphoreType.DMA,
    ],
    compiler_params=pltpu.CompilerParams(needs_layout_passes=False),
)
def _multi_hot(in_hbm, out_hbm, in0, in1, col0, col1, buf0, buf1,
               sin0, sin1, sout0, sout1):
    wid = lax.axis_index("s") * 2 + lax.axis_index("c")
    base_row = wid * _ROWS_PER_W
    lane = lax.iota(jnp.int32, 16)
    ones = jnp.ones((16,), jnp.float32)
    zeros = jnp.zeros((16,), jnp.float32)

    ins = (in0, in1)
    cols = (col0, col1)
    bufs = (buf0, buf1)
    sins = (sin0, sin1)
    souts = (sout0, sout1)

    def in_slice(c):
        return in_hbm.at[pl.ds((base_row + c * _CHUNK) * _SEQ, _CHUNK_ELEMS)]

    def out_slice(c):
        return out_hbm.at[pl.ds(base_row + c * _CHUNK, _CHUNK), :]

    def zero_row(r, carry):
        def zero_col(j, carry2):
            buf0[r, pl.ds(j * 16, 16)] = zeros
            buf1[r, pl.ds(j * 16, 16)] = zeros
            return carry2

        lax.fori_loop(0, _NUM_BINS // 16, zero_col, 0, unroll=8)
        buf0[r, pl.ds(_NUM_BINS - 16, 16)] = zeros
        buf1[r, pl.ds(_NUM_BINS - 16, 16)] = zeros
        return carry

    lax.fori_loop(0, _CHUNK, zero_row, 0)

    pltpu.async_copy(in_slice(0), in0, sin0)
    pltpu.async_copy(in_slice(1), in1, sin1)

    def compute(b, c):
        pltpu.make_async_copy(in_slice(c), ins[b], sins[b]).wait()

        def hash_body(i, e_vec):
            x = ins[b][pl.ds(i * 16, 16)]
            col = _hash16(x)
            row = e_vec // _SEQ
            plsc.store_scatter(bufs[b], [row, col], ones)
            cols[b][pl.ds(i * 16, 16)] = col
            return e_vec + 16

        lax.fori_loop(0, _NVEC, hash_body, lane, unroll=5)
        pltpu.async_copy(bufs[b], out_slice(c), souts[b])

        @pl.when(c + 2 < _CHUNKS)
        def _():
            pltpu.async_copy(in_slice(c + 2), ins[b], sins[b])

    compute(0, 0)
    compute(1, 1)

    def outer(i, carry):
        for b in (0, 1):
            c = i * 2 + b
            pltpu.make_async_copy(bufs[b], out_slice(c), souts[b]).wait()

            def reset_body(j, e_vec):
                col = cols[b][pl.ds(j * 16, 16)]
                row = e_vec // _SEQ
                plsc.store_scatter(bufs[b], [row, col], zeros)
                return e_vec + 16

            lax.fori_loop(0, _NVEC, reset_body, lane, unroll=5)
            compute(b, c)
        return carry

    lax.fori_loop(1, _CHUNKS // 2, outer, 0)

    pltpu.make_async_copy(buf0, out_slice(0), sout0).wait()
    pltpu.make_async_copy(buf1, out_slice(1), sout1).wait()


def kernel(inputs):
    flat = inputs.reshape(-1)
    out = _multi_hot(flat)
    return out.reshape(_NUM_FIELDS, _BATCH, _NUM_BINS)
